# Initial kernel scaffold; baseline (speedup 1.0000x reference)
import functools

import jax
import jax.numpy as jnp
from jax import lax
from jax.experimental import pallas as pl
from jax.experimental.pallas import tpu as pltpu
from jax.experimental.pallas import tpu_sc as plsc

N_USER = 50000
N_ITEM = 50000
N = 50000
E = 320000
D = 128
H = 128
L = 2
EPS = 1e-5

NC = 2
NS = 16
NW = NC * NS
EPW = E // NW
CSZ = 8192
NCHUNK = 7
CAP = EPW + NCHUNK * 16
ACC_ROWS = CSZ + 16
PN = NCHUNK * CSZ
ZROWS = ACC_ROWS // NS
OROWS = CSZ // NS

_i32 = jnp.int32
_f32 = jnp.float32


def _vmesh():
    return plsc.VectorSubcoreMesh(
        core_axis_name="c", subcore_axis_name="s", num_cores=NC, num_subcores=NS
    )


def _bin_kernel(src_ui, dst_ui, src_iu, dst_iu,
                bsrc_ui, bdst_ui, offs_ui, bsrc_iu, bdst_iu, offs_iu,
                src_v, dst_v, osrc_v, odst_v, offs_v, sem):
    wid = lax.axis_index("s") * NC + lax.axis_index("c")
    io16 = lax.iota(_i32, 16)
    for (srch, dsth, bsrch, bdsth, offsh) in (
        (src_ui, dst_ui, bsrc_ui, bdst_ui, offs_ui),
        (src_iu, dst_iu, bsrc_iu, bdst_iu, offs_iu),
    ):
        base = wid * EPW
        pltpu.async_copy(srch.at[pl.ds(base, EPW)], src_v, sem).wait()
        pltpu.async_copy(dsth.at[pl.ds(base, EPW)], dst_v, sem).wait()

        def cbody(i, accs):
            d = dst_v[pl.ds(i * 16, 16)]
            ch = lax.shift_right_logical(d, 13)
            return tuple(
                accs[b] + jnp.where(ch == b, 1, 0).astype(_i32)
                for b in range(NCHUNK)
            )
        accs = lax.fori_loop(
            0, EPW // 16, cbody,
            tuple(jnp.zeros((16,), _i32) for _ in range(NCHUNK)),
        )
        cnts = [jnp.sum(a) for a in accs]

        offs = [jnp.zeros((), _i32)]
        for b in range(NCHUNK):
            pcnt = lax.bitwise_and(cnts[b] + 15, jnp.full((), ~15, _i32))
            offs.append(offs[b] + pcnt)

        def fbody(i, pos):
            s = src_v[pl.ds(i * 16, 16)]
            d = dst_v[pl.ds(i * 16, 16)]
            ch = lax.shift_right_logical(d, 13)
            newpos = []
            for b in range(NCHUNK):
                m = ch == b
                mi = jnp.where(m, 1, 0).astype(_i32)
                posv = pos[b] + plsc.cumsum(mi) - 1
                plsc.store_scatter(osrc_v, [posv], s, mask=m)
                plsc.store_scatter(odst_v, [posv], d, mask=m)
                newpos.append(pos[b] + jnp.sum(mi))
            return tuple(newpos)
        pos = lax.fori_loop(0, EPW // 16, fbody, tuple(offs[:NCHUNK]))

        for b in range(NCHUNK):
            posv = pos[b] + io16
            m = posv < offs[b + 1]
            plsc.store_scatter(osrc_v, [posv], jnp.zeros((16,), _i32), mask=m)
            plsc.store_scatter(
                odst_v, [posv], jnp.full((16,), (b + 1) << 13, _i32), mask=m)

        ends = jnp.zeros((16,), _i32)
        for b in range(NCHUNK):
            ends = ends + jnp.where(io16 == b, offs[b + 1], 0).astype(_i32)
        offs_v[...] = ends

        pltpu.async_copy(osrc_v, bsrch.at[wid], sem).wait()
        pltpu.async_copy(odst_v, bdsth.at[wid], sem).wait()
        pltpu.async_copy(offs_v, offsh.at[wid], sem).wait()


def _bin_edges(src_ui, dst_ui, src_iu, dst_iu):
    out = jax.ShapeDtypeStruct
    k = pl.kernel(
        _bin_kernel,
        out_type=[
            out((NW, CAP), _i32), out((NW, CAP), _i32), out((NW, 16), _i32),
            out((NW, CAP), _i32), out((NW, CAP), _i32), out((NW, 16), _i32),
        ],
        mesh=_vmesh(),
        scratch_types=[
            pltpu.VMEM((EPW,), _i32), pltpu.VMEM((EPW,), _i32),
            pltpu.VMEM((CAP,), _i32), pltpu.VMEM((CAP,), _i32),
            pltpu.VMEM((16,), _i32), pltpu.SemaphoreType.DMA,
        ],
    )
    return k(src_ui, dst_ui, src_iu, dst_iu)


def _agg_kernel(with_deg, h_hbm, bsrc_hbm, bdst_hbm, offs_hbm, zeros_hbm,
                zeros16_hbm, *refs):
    if with_deg:
        part_hbm, dpart_hbm = refs[0], refs[1]
        scr = refs[2:]
    else:
        part_hbm = refs[0]
        dpart_hbm = None
        scr = refs[1:]
    (acc, dacc, src_v, dst_v, ldst_v, rows_v, ones_v,
     src_t, dst_t, ldst_t, rows_t, ones_t, offs_v, sem) = scr

    core = lax.axis_index("c")
    tl = lax.axis_index("s")
    wid = tl * NC + core
    io16 = lax.iota(_i32, 16)

    if with_deg:
        @pl.loop(0, 128)
        def _(i):
            ones_v[i, :] = jnp.ones((16,), _f32)

        @pl.loop(0, 16)
        def _(i):
            ones_t[i, :] = jnp.ones((16,), _f32)

    pltpu.async_copy(offs_hbm.at[wid], offs_v, sem).wait()
    ovec = offs_v[...]
    ends = [jnp.sum(jnp.where(io16 == b, ovec, 0).astype(_i32))
            for b in range(NCHUNK)]
    starts = [jnp.zeros((), _i32)] + ends[: NCHUNK - 1]

    for b in range(NCHUNK):
        plsc.subcore_barrier()
        pltpu.async_copy(zeros_hbm.at[pl.ds(tl * ZROWS, ZROWS)],
                         acc.at[pl.ds(tl * ZROWS, ZROWS)], sem).wait()
        if with_deg:
            pltpu.async_copy(zeros16_hbm.at[pl.ds(tl * ZROWS, ZROWS)],
                             dacc.at[pl.ds(tl * ZROWS, ZROWS)], sem).wait()
        plsc.subcore_barrier()

        s0 = starts[b]
        n = ends[b] - s0
        nmain = lax.bitwise_and(n, jnp.full((), ~127, _i32))
        cbase = jnp.full((), b << 13, _i32)

        def mbody(g, _):
            off = s0 + g * 128
            pltpu.async_copy(bsrc_hbm.at[wid, pl.ds(off, 128)], src_v,
                             sem).wait()
            pltpu.async_copy(bdst_hbm.at[wid, pl.ds(off, 128)], dst_v,
                             sem).wait()

            @pl.loop(0, 128, step=16)
            def _(k):
                ldst_v[pl.ds(k, 16)] = dst_v[pl.ds(k, 16)] - cbase

            pltpu.async_copy(h_hbm.at[src_v], rows_v, sem).wait()
            pltpu.sync_copy(rows_v, acc.at[ldst_v], add=True)
            if with_deg:
                pltpu.sync_copy(ones_v, dacc.at[ldst_v], add=True)
            return 0
        lax.fori_loop(0, nmain // 128, mbody, 0)

        def tbody(g, _):
            off = s0 + nmain + g * 16
            pltpu.async_copy(bsrc_hbm.at[wid, pl.ds(off, 16)], src_t,
                             sem).wait()
            pltpu.async_copy(bdst_hbm.at[wid, pl.ds(off, 16)], dst_t,
                             sem).wait()
            ldst_t[...] = dst_t[...] - cbase
            pltpu.async_copy(h_hbm.at[src_t], rows_t, sem).wait()
            pltpu.sync_copy(rows_t, acc.at[ldst_t], add=True)
            if with_deg:
                pltpu.sync_copy(ones_t, dacc.at[ldst_t], add=True)
            return 0
        lax.fori_loop(0, (n - nmain) // 16, tbody, 0)

        plsc.subcore_barrier()
        obase = b * CSZ + tl * OROWS
        pltpu.async_copy(acc.at[pl.ds(tl * OROWS, OROWS)],
                         part_hbm.at[core, pl.ds(obase, OROWS)], sem).wait()
        if with_deg:
            pltpu.async_copy(dacc.at[pl.ds(tl * OROWS, OROWS)],
                             dpart_hbm.at[core, pl.ds(obase, OROWS)],
                             sem).wait()


def _aggregate(h, bsrc, bdst, offs, zeros, zeros16, with_deg):
    out = jax.ShapeDtypeStruct
    outs = [out((NC, PN, D), _f32)]
    if with_deg:
        outs.append(out((NC, PN, 16), _f32))
    k = pl.kernel(
        functools.partial(_agg_kernel, with_deg),
        out_type=outs,
        mesh=_vmesh(),
        scratch_types=[
            pltpu.VMEM_SHARED((ACC_ROWS, D), _f32),
            pltpu.VMEM_SHARED((ACC_ROWS, 16), _f32),
            pltpu.VMEM((128,), _i32), pltpu.VMEM((128,), _i32),
            pltpu.VMEM((128,), _i32), pltpu.VMEM((128, D), _f32),
            pltpu.VMEM((128, 16), _f32),
            pltpu.VMEM((16,), _i32), pltpu.VMEM((16,), _i32),
            pltpu.VMEM((16,), _i32), pltpu.VMEM((16, D), _f32),
            pltpu.VMEM((16, 16), _f32),
            pltpu.VMEM((16,), _i32), pltpu.SemaphoreType.DMA,
        ],
    )
    res = k(h, bsrc, bdst, offs, zeros, zeros16)
    return res if with_deg else (res[0], None)


BLK = 1000
NBLK = N // BLK


def _t1_body(h_ref, p_ref, d_ref, ws_ref, wn_ref, b_ref, z_ref, st_ref):
    a = p_ref[0] + p_ref[1]
    deg = d_ref[0][:, 0:1] + d_ref[1][:, 0:1]
    hn = a * (1.0 / jnp.maximum(deg, 1.0))
    z = (jnp.dot(h_ref[...], ws_ref[...], preferred_element_type=_f32)
         + jnp.dot(hn, wn_ref[...], preferred_element_type=_f32)
         + b_ref[...])
    z_ref[...] = z
    zs = jnp.sum(z, axis=0, keepdims=True)
    zss = jnp.sum(z * z, axis=0, keepdims=True)
    i = pl.program_id(0)

    @pl.when(i == 0)
    def _():
        st_ref[0:1, :] = zs
        st_ref[1:2, :] = zss

    @pl.when(i > 0)
    def _():
        st_ref[0:1, :] = st_ref[0:1, :] + zs
        st_ref[1:2, :] = st_ref[1:2, :] + zss


def _t1(h, part, dpart, Ws, Wn, bias):
    return pl.pallas_call(
        _t1_body,
        grid=(NBLK,),
        in_specs=[
            pl.BlockSpec((BLK, D), lambda i: (i, 0)),
            pl.BlockSpec((NC, BLK, D), lambda i: (0, i, 0)),
            pl.BlockSpec((NC, BLK, 16), lambda i: (0, i, 0)),
            pl.BlockSpec((D, H), lambda i: (0, 0)),
            pl.BlockSpec((D, H), lambda i: (0, 0)),
            pl.BlockSpec((1, H), lambda i: (0, 0)),
        ],
        out_specs=[
            pl.BlockSpec((BLK, H), lambda i: (i, 0)),
            pl.BlockSpec((2, H), lambda i: (0, 0)),
        ],
        out_shape=[
            jax.ShapeDtypeStruct((N, H), _f32),
            jax.ShapeDtypeStruct((2, H), _f32),
        ],
    )(h, part, dpart, Ws, Wn, bias.reshape(1, H))


def _t2_body(final, z_ref, st_ref, g_ref, be_ref, pw_ref, pb_ref, o_ref):
    mu = st_ref[0:1, :] * (1.0 / N)
    var = st_ref[1:2, :] * (1.0 / N) - mu * mu
    scale = g_ref[...] * lax.rsqrt(var + EPS)
    hnew = jnp.maximum((z_ref[...] - mu) * scale + be_ref[...], 0.0)
    if final:
        o_ref[...] = (jnp.dot(hnew, pw_ref[...], preferred_element_type=_f32)
                      + pb_ref[...])
    else:
        o_ref[...] = hnew


def _t2(z, st, gamma, beta, pW=None, pb=None):
    final = pW is not None
    if not final:
        pW = jnp.zeros((H, D), _f32)
        pb = jnp.zeros((D,), _f32)
    return pl.pallas_call(
        functools.partial(_t2_body, final),
        grid=(NBLK,),
        in_specs=[
            pl.BlockSpec((BLK, H), lambda i: (i, 0)),
            pl.BlockSpec((2, H), lambda i: (0, 0)),
            pl.BlockSpec((1, H), lambda i: (0, 0)),
            pl.BlockSpec((1, H), lambda i: (0, 0)),
            pl.BlockSpec((H, D), lambda i: (0, 0)),
            pl.BlockSpec((1, D), lambda i: (0, 0)),
        ],
        out_specs=pl.BlockSpec((BLK, D), lambda i: (i, 0)),
        out_shape=jax.ShapeDtypeStruct((N, D), _f32),
    )(z, st, gamma.reshape(1, H), beta.reshape(1, H), pW, pb.reshape(1, D))


def kernel(x_user, x_item, Wself, Wneigh, b, bn_gamma, bn_beta, proj_W,
           proj_b, edge_ui, edge_iu):
    src_ui = edge_ui[0].astype(_i32)
    dst_ui = edge_ui[1].astype(_i32)
    src_iu = edge_iu[0].astype(_i32)
    dst_iu = edge_iu[1].astype(_i32)

    (bsrc_ui, bdst_ui, offs_ui, bsrc_iu, bdst_iu, offs_iu) = _bin_edges(
        src_ui, dst_ui, src_iu, dst_iu)

    zeros = jnp.zeros((ACC_ROWS, D), _f32)
    zeros16 = jnp.zeros((ACC_ROWS, 16), _f32)

    hu, hi = x_user, x_item
    dpart_ui = dpart_iu = None
    out_u = out_i = None
    for i in range(L):
        with_deg = i == 0
        part_ui, dp_ui = _aggregate(hu, bsrc_ui, bdst_ui, offs_ui, zeros,
                                    zeros16, with_deg)
        part_iu, dp_iu = _aggregate(hi, bsrc_iu, bdst_iu, offs_iu, zeros,
                                    zeros16, with_deg)
        if with_deg:
            dpart_ui, dpart_iu = dp_ui, dp_iu
        z_i, st_i = _t1(hi, part_ui, dpart_ui, Wself[i, 0], Wneigh[i, 0],
                        b[i, 0])
        z_u, st_u = _t1(hu, part_iu, dpart_iu, Wself[i, 1], Wneigh[i, 1],
                        b[i, 1])
        if i < L - 1:
            hu = _t2(z_u, st_u, bn_gamma[i, 0], bn_beta[i, 0])
            hi = _t2(z_i, st_i, bn_gamma[i, 1], bn_beta[i, 1])
        else:
            out_u = _t2(z_u, st_u, bn_gamma[i, 0], bn_beta[i, 0],
                        proj_W[0], proj_b[0])
            out_i = _t2(z_i, st_i, bn_gamma[i, 1], bn_beta[i, 1],
                        proj_W[1], proj_b[1])
    return (out_u, out_i)


# speedup vs baseline: 2.6259x; 2.6259x over previous
import dataclasses
import functools

import jax
import jax.numpy as jnp
from jax import lax
from jax.experimental import pallas as pl
from jax.experimental.pallas import tpu as pltpu
from jax.experimental.pallas import tpu_sc as plsc

N_USER = 50000
N_ITEM = 50000
N = 50000
E = 320000
D = 128
H = 128
L = 2
EPS = 1e-5

NC = 2
NS = 16
NW = NC * NS
EPW = E // NW
CSZ = 4096
SHIFT = 12
NCHUNK = 13
CAP = EPW + NCHUNK * 16
ACC_ROWS = CSZ + 128
PN = NCHUNK * CSZ
ZROWS = ACC_ROWS // NS
OROWS = CSZ // NS

_i32 = jnp.int32
_f32 = jnp.float32


def _vmesh():
    return plsc.VectorSubcoreMesh(
        core_axis_name="c", subcore_axis_name="s", num_cores=NC, num_subcores=NS
    )


def _sc_params():
    cp = pltpu.CompilerParams()
    if "needs_layout_passes" in pltpu.CompilerParams.__dataclass_fields__:
        cp = dataclasses.replace(cp, needs_layout_passes=False)
    return cp


def _bin_kernel(src_ui, dst_ui, src_iu, dst_iu,
                bsrc_ui, bdst_ui, offs_ui, bsrc_iu, bdst_iu, offs_iu,
                src_v, dst_v, osrc_v, odst_v, offs_v, sem):
    wid = lax.axis_index("s") * NC + lax.axis_index("c")
    io16 = lax.iota(_i32, 16)
    for (srch, dsth, bsrch, bdsth, offsh) in (
        (src_ui, dst_ui, bsrc_ui, bdst_ui, offs_ui),
        (src_iu, dst_iu, bsrc_iu, bdst_iu, offs_iu),
    ):
        base = wid * EPW
        pltpu.async_copy(srch.at[pl.ds(base, EPW)], src_v, sem).wait()
        pltpu.async_copy(dsth.at[pl.ds(base, EPW)], dst_v, sem).wait()

        def cbody(i, accs):
            d = dst_v[pl.ds(i * 16, 16)]
            ch = lax.shift_right_logical(d, SHIFT)
            return tuple(
                accs[b] + jnp.where(ch == b, 1, 0).astype(_i32)
                for b in range(NCHUNK)
            )
        accs = lax.fori_loop(
            0, EPW // 16, cbody,
            tuple(jnp.zeros((16,), _i32) for _ in range(NCHUNK)),
        )
        cnts = [jnp.sum(a) for a in accs]

        offs = [jnp.zeros((), _i32)]
        for b in range(NCHUNK):
            pcnt = lax.bitwise_and(cnts[b] + 15, jnp.full((), ~15, _i32))
            offs.append(offs[b] + pcnt)

        def fbody(i, pos):
            s = src_v[pl.ds(i * 16, 16)]
            d = dst_v[pl.ds(i * 16, 16)]
            ch = lax.shift_right_logical(d, SHIFT)
            newpos = []
            for b in range(NCHUNK):
                m = ch == b
                mi = jnp.where(m, 1, 0).astype(_i32)
                posv = pos[b] + plsc.cumsum(mi) - 1
                posv = jnp.clip(posv, 0, CAP - 1)
                plsc.store_scatter(osrc_v, [posv], s, mask=m)
                plsc.store_scatter(odst_v, [posv], d, mask=m)
                newpos.append(pos[b] + jnp.sum(mi))
            return tuple(newpos)
        pos = lax.fori_loop(0, EPW // 16, fbody, tuple(offs[:NCHUNK]))

        for b in range(NCHUNK):
            posv = pos[b] + io16
            m = posv < offs[b + 1]
            posv = jnp.clip(posv, 0, CAP - 1)
            plsc.store_scatter(osrc_v, [posv], jnp.zeros((16,), _i32), mask=m)
            plsc.store_scatter(
                odst_v, [posv], jnp.full((16,), (b + 1) * CSZ, _i32), mask=m)

        ends = jnp.zeros((16,), _i32)
        for b in range(NCHUNK):
            ends = ends + jnp.where(io16 == b, offs[b + 1], 0).astype(_i32)
        offs_v[...] = ends

        pltpu.async_copy(osrc_v, bsrch.at[pl.ds(wid * CAP, CAP)], sem).wait()
        pltpu.async_copy(odst_v, bdsth.at[pl.ds(wid * CAP, CAP)], sem).wait()
        pltpu.async_copy(offs_v, offsh.at[pl.ds(wid * 16, 16)], sem).wait()


def _bin_edges(src_ui, dst_ui, src_iu, dst_iu):
    out = jax.ShapeDtypeStruct
    k = pl.kernel(
        _bin_kernel,
        out_type=[
            out((NW * CAP,), _i32), out((NW * CAP,), _i32),
            out((NW * 16,), _i32),
            out((NW * CAP,), _i32), out((NW * CAP,), _i32),
            out((NW * 16,), _i32),
        ],
        mesh=_vmesh(),
        scratch_types=[
            pltpu.VMEM((EPW,), _i32), pltpu.VMEM((EPW,), _i32),
            pltpu.VMEM((CAP,), _i32), pltpu.VMEM((CAP,), _i32),
            pltpu.VMEM((16,), _i32), pltpu.SemaphoreType.DMA,
        ],
        compiler_params=_sc_params(),
    )
    return k(src_ui, dst_ui, src_iu, dst_iu)


def _agg_kernel(with_deg, h_hbm, bsrc_hbm, bdst_hbm, offs_hbm, zeros_hbm,
                ones_hbm, *refs):
    if with_deg:
        part_hbm, dpart_hbm = refs[0], refs[1]
        scr = refs[2:]
    else:
        part_hbm = refs[0]
        dpart_hbm = None
        scr = refs[1:]
    (acc, dacc, src_v, dst_v, ldst_v, rows_v, ones_v,
     src_t, dst_t, ldst_t, rows_t, ones_t, offs_v, sem) = scr

    core = lax.axis_index("c")
    tl = lax.axis_index("s")
    wid = tl * NC + core
    io16 = lax.iota(_i32, 16)

    if with_deg:
        pltpu.sync_copy(ones_hbm, ones_v)
        pltpu.sync_copy(ones_hbm.at[pl.ds(0, 16)], ones_t)

    pltpu.async_copy(offs_hbm.at[pl.ds(wid * 16, 16)], offs_v, sem).wait()
    ovec = offs_v[...]
    ends = [jnp.sum(jnp.where(io16 == b, ovec, 0).astype(_i32))
            for b in range(NCHUNK)]
    starts = [jnp.zeros((), _i32)] + ends[: NCHUNK - 1]

    for b in range(NCHUNK):
        plsc.subcore_barrier()
        pltpu.async_copy(zeros_hbm.at[pl.ds(tl * ZROWS, ZROWS)],
                         acc.at[pl.ds(tl * ZROWS, ZROWS)], sem).wait()
        if with_deg:
            pltpu.async_copy(zeros_hbm.at[pl.ds(tl * ZROWS, ZROWS)],
                             dacc.at[pl.ds(tl * ZROWS, ZROWS)], sem).wait()
        plsc.subcore_barrier()

        s0 = starts[b]
        n = ends[b] - s0
        nmain = lax.bitwise_and(n, jnp.full((), ~127, _i32))
        cbase = jnp.full((), b * CSZ, _i32)

        def mbody(g, _):
            off = s0 + g * 128
            eoff = pl.multiple_of(wid * CAP + off, 16)
            pltpu.async_copy(bsrc_hbm.at[pl.ds(eoff, 128)], src_v,
                             sem).wait()
            pltpu.async_copy(bdst_hbm.at[pl.ds(eoff, 128)], dst_v,
                             sem).wait()

            @pl.loop(0, 128, step=16)
            def _(k):
                ldst_v[pl.ds(k, 16)] = dst_v[pl.ds(k, 16)] - cbase

            pltpu.async_copy(h_hbm.at[src_v], rows_v, sem).wait()
            pltpu.sync_copy(rows_v, acc.at[ldst_v], add=True)
            if with_deg:
                pltpu.sync_copy(ones_v, dacc.at[ldst_v], add=True)
            return 0
        lax.fori_loop(0, nmain // 128, mbody, 0)

        def tbody(g, _):
            off = s0 + nmain + g * 16
            eoff = pl.multiple_of(wid * CAP + off, 16)
            pltpu.async_copy(bsrc_hbm.at[pl.ds(eoff, 16)], src_t,
                             sem).wait()
            pltpu.async_copy(bdst_hbm.at[pl.ds(eoff, 16)], dst_t,
                             sem).wait()
            ldst_t[...] = dst_t[...] - cbase
            pltpu.async_copy(h_hbm.at[src_t], rows_t, sem).wait()
            pltpu.sync_copy(rows_t, acc.at[ldst_t], add=True)
            if with_deg:
                pltpu.sync_copy(ones_t, dacc.at[ldst_t], add=True)
            return 0
        lax.fori_loop(0, (n - nmain) // 16, tbody, 0)

        plsc.subcore_barrier()
        obase = b * CSZ + tl * OROWS
        pltpu.async_copy(acc.at[pl.ds(tl * OROWS, OROWS)],
                         part_hbm.at[core, pl.ds(obase, OROWS)], sem).wait()
        if with_deg:
            pltpu.async_copy(dacc.at[pl.ds(tl * OROWS, OROWS)],
                             dpart_hbm.at[core, pl.ds(obase, OROWS)],
                             sem).wait()


def _aggregate(h, bsrc, bdst, offs, zeros, ones, with_deg):
    out = jax.ShapeDtypeStruct
    outs = [out((NC, PN, D), _f32)]
    if with_deg:
        outs.append(out((NC, PN, D), _f32))
    k = pl.kernel(
        functools.partial(_agg_kernel, with_deg),
        out_type=outs,
        mesh=_vmesh(),
        scratch_types=[
            pltpu.VMEM_SHARED((ACC_ROWS, D), _f32),
            pltpu.VMEM_SHARED((ACC_ROWS, D), _f32),
            pltpu.VMEM((128,), _i32), pltpu.VMEM((128,), _i32),
            pltpu.VMEM((128,), _i32), pltpu.VMEM((128, D), _f32),
            pltpu.VMEM((128, D), _f32),
            pltpu.VMEM((16,), _i32), pltpu.VMEM((16,), _i32),
            pltpu.VMEM((16,), _i32), pltpu.VMEM((16, D), _f32),
            pltpu.VMEM((16, D), _f32),
            pltpu.VMEM((16,), _i32), pltpu.SemaphoreType.DMA,
        ],
        compiler_params=_sc_params(),
    )
    res = k(h, bsrc, bdst, offs, zeros, ones)
    return res if with_deg else (res[0], None)


BLK = 1000
NBLK = N // BLK


def _t1_body(h_ref, p_ref, d_ref, ws_ref, wn_ref, b_ref, z_ref, st_ref):
    a = p_ref[0] + p_ref[1]
    deg = d_ref[0][:, 0:1] + d_ref[1][:, 0:1]
    hn = a * (1.0 / jnp.maximum(deg, 1.0))
    z = (jnp.dot(h_ref[...], ws_ref[...], preferred_element_type=_f32)
         + jnp.dot(hn, wn_ref[...], preferred_element_type=_f32)
         + b_ref[...])
    z_ref[...] = z
    zs = jnp.sum(z, axis=0, keepdims=True)
    zss = jnp.sum(z * z, axis=0, keepdims=True)
    i = pl.program_id(0)

    @pl.when(i == 0)
    def _():
        st_ref[0:1, :] = zs
        st_ref[1:2, :] = zss

    @pl.when(i > 0)
    def _():
        st_ref[0:1, :] = st_ref[0:1, :] + zs
        st_ref[1:2, :] = st_ref[1:2, :] + zss


def _t1(h, part, dpart, Ws, Wn, bias):
    return pl.pallas_call(
        _t1_body,
        grid=(NBLK,),
        in_specs=[
            pl.BlockSpec((BLK, D), lambda i: (i, 0)),
            pl.BlockSpec((NC, BLK, D), lambda i: (0, i, 0)),
            pl.BlockSpec((NC, BLK, D), lambda i: (0, i, 0)),
            pl.BlockSpec((D, H), lambda i: (0, 0)),
            pl.BlockSpec((D, H), lambda i: (0, 0)),
            pl.BlockSpec((1, H), lambda i: (0, 0)),
        ],
        out_specs=[
            pl.BlockSpec((BLK, H), lambda i: (i, 0)),
            pl.BlockSpec((2, H), lambda i: (0, 0)),
        ],
        out_shape=[
            jax.ShapeDtypeStruct((N, H), _f32),
            jax.ShapeDtypeStruct((2, H), _f32),
        ],
    )(h, part, dpart, Ws, Wn, bias.reshape(1, H))


def _t2_body(final, z_ref, st_ref, g_ref, be_ref, pw_ref, pb_ref, o_ref):
    mu = st_ref[0:1, :] * (1.0 / N)
    var = st_ref[1:2, :] * (1.0 / N) - mu * mu
    scale = g_ref[...] * lax.rsqrt(var + EPS)
    hnew = jnp.maximum((z_ref[...] - mu) * scale + be_ref[...], 0.0)
    if final:
        o_ref[...] = (jnp.dot(hnew, pw_ref[...], preferred_element_type=_f32)
                      + pb_ref[...])
    else:
        o_ref[...] = hnew


def _t2(z, st, gamma, beta, pW=None, pb=None):
    final = pW is not None
    if not final:
        pW = jnp.zeros((H, D), _f32)
        pb = jnp.zeros((D,), _f32)
    return pl.pallas_call(
        functools.partial(_t2_body, final),
        grid=(NBLK,),
        in_specs=[
            pl.BlockSpec((BLK, H), lambda i: (i, 0)),
            pl.BlockSpec((2, H), lambda i: (0, 0)),
            pl.BlockSpec((1, H), lambda i: (0, 0)),
            pl.BlockSpec((1, H), lambda i: (0, 0)),
            pl.BlockSpec((H, D), lambda i: (0, 0)),
            pl.BlockSpec((1, D), lambda i: (0, 0)),
        ],
        out_specs=pl.BlockSpec((BLK, D), lambda i: (i, 0)),
        out_shape=jax.ShapeDtypeStruct((N, D), _f32),
    )(z, st, gamma.reshape(1, H), beta.reshape(1, H), pW, pb.reshape(1, D))


def kernel(x_user, x_item, Wself, Wneigh, b, bn_gamma, bn_beta, proj_W,
           proj_b, edge_ui, edge_iu):
    src_ui = edge_ui[0].astype(_i32)
    dst_ui = edge_ui[1].astype(_i32)
    src_iu = edge_iu[0].astype(_i32)
    dst_iu = edge_iu[1].astype(_i32)

    (bsrc_ui, bdst_ui, offs_ui, bsrc_iu, bdst_iu, offs_iu) = _bin_edges(
        src_ui, dst_ui, src_iu, dst_iu)

    zeros = jnp.zeros((ACC_ROWS, D), _f32)
    ones = jnp.ones((128, D), _f32)

    hu, hi = x_user, x_item
    dpart_ui = dpart_iu = None
    out_u = out_i = None
    for i in range(L):
        with_deg = i == 0
        part_ui, dp_ui = _aggregate(hu, bsrc_ui, bdst_ui, offs_ui, zeros,
                                    ones, with_deg)
        part_iu, dp_iu = _aggregate(hi, bsrc_iu, bdst_iu, offs_iu, zeros,
                                    ones, with_deg)
        if with_deg:
            dpart_ui, dpart_iu = dp_ui, dp_iu
        z_i, st_i = _t1(hi, part_ui, dpart_ui, Wself[i, 0], Wneigh[i, 0],
                        b[i, 0])
        z_u, st_u = _t1(hu, part_iu, dpart_iu, Wself[i, 1], Wneigh[i, 1],
                        b[i, 1])
        if i < L - 1:
            hu = _t2(z_u, st_u, bn_gamma[i, 0], bn_beta[i, 0])
            hi = _t2(z_i, st_i, bn_gamma[i, 1], bn_beta[i, 1])
        else:
            out_u = _t2(z_u, st_u, bn_gamma[i, 0], bn_beta[i, 0],
                        proj_W[0], proj_b[0])
            out_i = _t2(z_i, st_i, bn_gamma[i, 1], bn_beta[i, 1],
                        proj_W[1], proj_b[1])
    return (out_u, out_i)


# speedup vs baseline: 2.9884x; 1.1380x over previous
import dataclasses
import functools

import jax
import jax.numpy as jnp
from jax import lax
from jax.experimental import pallas as pl
from jax.experimental.pallas import tpu as pltpu
from jax.experimental.pallas import tpu_sc as plsc

N_USER = 50000
N_ITEM = 50000
N = 50000
E = 320000
D = 128
H = 128
L = 2
EPS = 1e-5

NC = 2
NS = 16
NW = NC * NS
EPW = E // NW
CSZ = 4096
SHIFT = 12
NCHUNK = 13
CAP = EPW + NCHUNK * 16
ACC_ROWS = CSZ + 128
PN = NCHUNK * CSZ
ZROWS = ACC_ROWS // NS
OROWS = CSZ // NS

_i32 = jnp.int32
_f32 = jnp.float32


def _vmesh():
    return plsc.VectorSubcoreMesh(
        core_axis_name="c", subcore_axis_name="s", num_cores=NC, num_subcores=NS
    )


def _sc_params():
    cp = pltpu.CompilerParams()
    if "needs_layout_passes" in pltpu.CompilerParams.__dataclass_fields__:
        cp = dataclasses.replace(cp, needs_layout_passes=False)
    return cp


def _bin_kernel(src_ui, dst_ui, src_iu, dst_iu,
                bsrc_ui, bdst_ui, offs_ui, bsrc_iu, bdst_iu, offs_iu,
                src_v, dst_v, osrc_v, odst_v, offs_v, sem):
    wid = lax.axis_index("s") * NC + lax.axis_index("c")
    io16 = lax.iota(_i32, 16)
    for (srch, dsth, bsrch, bdsth, offsh) in (
        (src_ui, dst_ui, bsrc_ui, bdst_ui, offs_ui),
        (src_iu, dst_iu, bsrc_iu, bdst_iu, offs_iu),
    ):
        base = wid * EPW
        pltpu.async_copy(srch.at[pl.ds(base, EPW)], src_v, sem).wait()
        pltpu.async_copy(dsth.at[pl.ds(base, EPW)], dst_v, sem).wait()

        def cbody(i, accs):
            d = dst_v[pl.ds(i * 16, 16)]
            ch = lax.shift_right_logical(d, SHIFT)
            return tuple(
                accs[b] + jnp.where(ch == b, 1, 0).astype(_i32)
                for b in range(NCHUNK)
            )
        accs = lax.fori_loop(
            0, EPW // 16, cbody,
            tuple(jnp.zeros((16,), _i32) for _ in range(NCHUNK)),
        )
        cnts = [jnp.sum(a) for a in accs]

        offs = [jnp.zeros((), _i32)]
        for b in range(NCHUNK):
            pcnt = lax.bitwise_and(cnts[b] + 15, jnp.full((), ~15, _i32))
            offs.append(offs[b] + pcnt)

        def fbody(i, pos):
            s = src_v[pl.ds(i * 16, 16)]
            d = dst_v[pl.ds(i * 16, 16)]
            ch = lax.shift_right_logical(d, SHIFT)
            newpos = []
            for b in range(NCHUNK):
                m = ch == b
                mi = jnp.where(m, 1, 0).astype(_i32)
                posv = pos[b] + plsc.cumsum(mi) - 1
                posv = jnp.clip(posv, 0, CAP - 1)
                plsc.store_scatter(osrc_v, [posv], s, mask=m)
                plsc.store_scatter(odst_v, [posv], d, mask=m)
                newpos.append(pos[b] + jnp.sum(mi))
            return tuple(newpos)
        pos = lax.fori_loop(0, EPW // 16, fbody, tuple(offs[:NCHUNK]))

        for b in range(NCHUNK):
            posv = pos[b] + io16
            m = posv < offs[b + 1]
            posv = jnp.clip(posv, 0, CAP - 1)
            plsc.store_scatter(osrc_v, [posv], jnp.zeros((16,), _i32), mask=m)
            plsc.store_scatter(
                odst_v, [posv], jnp.full((16,), (b + 1) * CSZ, _i32), mask=m)

        ends = jnp.zeros((16,), _i32)
        for b in range(NCHUNK):
            ends = ends + jnp.where(io16 == b, offs[b + 1], 0).astype(_i32)
        offs_v[...] = ends

        pltpu.async_copy(osrc_v, bsrch.at[pl.ds(wid * CAP, CAP)], sem).wait()
        pltpu.async_copy(odst_v, bdsth.at[pl.ds(wid * CAP, CAP)], sem).wait()
        pltpu.async_copy(offs_v, offsh.at[pl.ds(wid * 16, 16)], sem).wait()


def _bin_edges(src_ui, dst_ui, src_iu, dst_iu):
    out = jax.ShapeDtypeStruct
    k = pl.kernel(
        _bin_kernel,
        out_type=[
            out((NW * CAP,), _i32), out((NW * CAP,), _i32),
            out((NW * 16,), _i32),
            out((NW * CAP,), _i32), out((NW * CAP,), _i32),
            out((NW * 16,), _i32),
        ],
        mesh=_vmesh(),
        scratch_types=[
            pltpu.VMEM((EPW,), _i32), pltpu.VMEM((EPW,), _i32),
            pltpu.VMEM((CAP,), _i32), pltpu.VMEM((CAP,), _i32),
            pltpu.VMEM((16,), _i32), pltpu.SemaphoreType.DMA,
        ],
        compiler_params=_sc_params(),
    )
    return k(src_ui, dst_ui, src_iu, dst_iu)


def _agg_kernel(with_deg, h_hbm, bsrc_hbm, bdst_hbm, offs_hbm, zeros_hbm,
                ones_hbm, *refs):
    if with_deg:
        part_hbm, dpart_hbm = refs[0], refs[1]
        scr = refs[2:]
    else:
        part_hbm = refs[0]
        dpart_hbm = None
        scr = refs[1:]
    (acc, dacc, src_a, src_b, dst_a, dst_b, ldst_a, ldst_b, rows_a, rows_b,
     ones_v, ldst_t, rows_t, ones_t, offs_v, sem, sem2, sem3, sem4) = scr

    core = lax.axis_index("c")
    tl = lax.axis_index("s")
    wid = tl * NC + core
    io16 = lax.iota(_i32, 16)

    if with_deg:
        pltpu.sync_copy(ones_hbm, ones_v)
        pltpu.sync_copy(ones_hbm.at[pl.ds(0, 16)], ones_t)

    pltpu.async_copy(offs_hbm.at[pl.ds(wid * 16, 16)], offs_v, sem).wait()
    ovec = offs_v[...]
    ends = [jnp.sum(jnp.where(io16 == b, ovec, 0).astype(_i32))
            for b in range(NCHUNK)]
    starts = [jnp.zeros((), _i32)] + ends[: NCHUNK - 1]

    for b in range(NCHUNK):
        plsc.subcore_barrier()
        pltpu.async_copy(zeros_hbm.at[pl.ds(tl * ZROWS, ZROWS)],
                         acc.at[pl.ds(tl * ZROWS, ZROWS)], sem).wait()
        if with_deg:
            pltpu.async_copy(zeros_hbm.at[pl.ds(tl * ZROWS, ZROWS)],
                             dacc.at[pl.ds(tl * ZROWS, ZROWS)], sem).wait()
        plsc.subcore_barrier()

        s0 = starts[b]
        n = ends[b] - s0
        npair = lax.bitwise_and(n, jnp.full((), ~255, _i32))
        nmain = lax.bitwise_and(n, jnp.full((), ~127, _i32))
        cbase = jnp.full((), b * CSZ, _i32)

        def pbody(g, _):
            offa = pl.multiple_of(wid * CAP + s0 + g * 256, 16)
            offb = pl.multiple_of(offa + 128, 16)
            ia = pltpu.async_copy(bsrc_hbm.at[pl.ds(offa, 128)], src_a, sem)
            ib = pltpu.async_copy(bsrc_hbm.at[pl.ds(offb, 128)], src_b, sem2)
            da = pltpu.async_copy(bdst_hbm.at[pl.ds(offa, 128)], dst_a, sem3)
            db = pltpu.async_copy(bdst_hbm.at[pl.ds(offb, 128)], dst_b, sem4)
            ia.wait()
            ga = pltpu.async_copy(h_hbm.at[src_a], rows_a, sem)
            ib.wait()
            gb = pltpu.async_copy(h_hbm.at[src_b], rows_b, sem2)
            da.wait()
            db.wait()

            @pl.loop(0, 128, step=16)
            def _(k):
                ldst_a[pl.ds(k, 16)] = dst_a[pl.ds(k, 16)] - cbase
                ldst_b[pl.ds(k, 16)] = dst_b[pl.ds(k, 16)] - cbase

            ga.wait()
            pltpu.sync_copy(rows_a, acc.at[ldst_a], add=True)
            if with_deg:
                pltpu.sync_copy(ones_v, dacc.at[ldst_a], add=True)
            gb.wait()
            pltpu.sync_copy(rows_b, acc.at[ldst_b], add=True)
            if with_deg:
                pltpu.sync_copy(ones_v, dacc.at[ldst_b], add=True)
            return 0
        lax.fori_loop(0, npair // 256, pbody, 0)

        @pl.when(nmain > npair)
        def _():
            offa = pl.multiple_of(wid * CAP + s0 + npair, 16)
            pltpu.async_copy(bsrc_hbm.at[pl.ds(offa, 128)], src_a, sem).wait()
            pltpu.async_copy(bdst_hbm.at[pl.ds(offa, 128)], dst_a, sem).wait()

            @pl.loop(0, 128, step=16)
            def _(k):
                ldst_a[pl.ds(k, 16)] = dst_a[pl.ds(k, 16)] - cbase

            pltpu.async_copy(h_hbm.at[src_a], rows_a, sem).wait()
            pltpu.sync_copy(rows_a, acc.at[ldst_a], add=True)
            if with_deg:
                pltpu.sync_copy(ones_v, dacc.at[ldst_a], add=True)

        def tbody(g, _):
            off = pl.multiple_of(wid * CAP + s0 + nmain + g * 16, 16)
            pltpu.async_copy(bsrc_hbm.at[pl.ds(off, 16)], src_a.at[pl.ds(0, 16)],
                             sem).wait()
            pltpu.async_copy(bdst_hbm.at[pl.ds(off, 16)], dst_a.at[pl.ds(0, 16)],
                             sem).wait()
            ldst_t[...] = dst_a[pl.ds(0, 16)] - cbase
            pltpu.async_copy(h_hbm.at[src_a.at[pl.ds(0, 16)]], rows_t,
                             sem).wait()
            pltpu.sync_copy(rows_t, acc.at[ldst_t], add=True)
            if with_deg:
                pltpu.sync_copy(ones_t, dacc.at[ldst_t], add=True)
            return 0
        lax.fori_loop(0, (n - nmain) // 16, tbody, 0)

        plsc.subcore_barrier()
        obase = b * CSZ + tl * OROWS
        pltpu.async_copy(acc.at[pl.ds(tl * OROWS, OROWS)],
                         part_hbm.at[core, pl.ds(obase, OROWS)], sem).wait()
        if with_deg:
            pltpu.async_copy(dacc.at[pl.ds(tl * OROWS, OROWS)],
                             dpart_hbm.at[core, pl.ds(obase, OROWS)],
                             sem).wait()


def _aggregate(h, bsrc, bdst, offs, zeros, ones, with_deg):
    out = jax.ShapeDtypeStruct
    outs = [out((NC, PN, D), _f32)]
    if with_deg:
        outs.append(out((NC, PN, D), _f32))
    k = pl.kernel(
        functools.partial(_agg_kernel, with_deg),
        out_type=outs,
        mesh=_vmesh(),
        scratch_types=[
            pltpu.VMEM_SHARED((ACC_ROWS, D), _f32),
            pltpu.VMEM_SHARED((ACC_ROWS, D), _f32),
            pltpu.VMEM((128,), _i32), pltpu.VMEM((128,), _i32),
            pltpu.VMEM((128,), _i32), pltpu.VMEM((128,), _i32),
            pltpu.VMEM((128,), _i32), pltpu.VMEM((128,), _i32),
            pltpu.VMEM((128, D), _f32), pltpu.VMEM((128, D), _f32),
            pltpu.VMEM((128, D), _f32),
            pltpu.VMEM((16,), _i32), pltpu.VMEM((16, D), _f32),
            pltpu.VMEM((16, D), _f32),
            pltpu.VMEM((16,), _i32),
            pltpu.SemaphoreType.DMA, pltpu.SemaphoreType.DMA,
            pltpu.SemaphoreType.DMA, pltpu.SemaphoreType.DMA,
        ],
        compiler_params=_sc_params(),
    )
    res = k(h, bsrc, bdst, offs, zeros, ones)
    return res if with_deg else (res[0], None)


BLK = 1000
NBLK = N // BLK


def _t1_body(h_ref, p_ref, d_ref, ws_ref, wn_ref, b_ref, z_ref, st_ref):
    a = p_ref[0] + p_ref[1]
    deg = d_ref[0][:, 0:1] + d_ref[1][:, 0:1]
    hn = a * (1.0 / jnp.maximum(deg, 1.0))
    z = (jnp.dot(h_ref[...], ws_ref[...], preferred_element_type=_f32)
         + jnp.dot(hn, wn_ref[...], preferred_element_type=_f32)
         + b_ref[...])
    z_ref[...] = z
    zs = jnp.sum(z, axis=0, keepdims=True)
    zss = jnp.sum(z * z, axis=0, keepdims=True)
    i = pl.program_id(0)

    @pl.when(i == 0)
    def _():
        st_ref[0:1, :] = zs
        st_ref[1:2, :] = zss

    @pl.when(i > 0)
    def _():
        st_ref[0:1, :] = st_ref[0:1, :] + zs
        st_ref[1:2, :] = st_ref[1:2, :] + zss


def _t1(h, part, dpart, Ws, Wn, bias):
    return pl.pallas_call(
        _t1_body,
        grid=(NBLK,),
        in_specs=[
            pl.BlockSpec((BLK, D), lambda i: (i, 0)),
            pl.BlockSpec((NC, BLK, D), lambda i: (0, i, 0)),
            pl.BlockSpec((NC, BLK, D), lambda i: (0, i, 0)),
            pl.BlockSpec((D, H), lambda i: (0, 0)),
            pl.BlockSpec((D, H), lambda i: (0, 0)),
            pl.BlockSpec((1, H), lambda i: (0, 0)),
        ],
        out_specs=[
            pl.BlockSpec((BLK, H), lambda i: (i, 0)),
            pl.BlockSpec((2, H), lambda i: (0, 0)),
        ],
        out_shape=[
            jax.ShapeDtypeStruct((N, H), _f32),
            jax.ShapeDtypeStruct((2, H), _f32),
        ],
    )(h, part, dpart, Ws, Wn, bias.reshape(1, H))


def _t2_body(final, z_ref, st_ref, g_ref, be_ref, pw_ref, pb_ref, o_ref):
    mu = st_ref[0:1, :] * (1.0 / N)
    var = st_ref[1:2, :] * (1.0 / N) - mu * mu
    scale = g_ref[...] * lax.rsqrt(var + EPS)
    hnew = jnp.maximum((z_ref[...] - mu) * scale + be_ref[...], 0.0)
    if final:
        o_ref[...] = (jnp.dot(hnew, pw_ref[...], preferred_element_type=_f32)
                      + pb_ref[...])
    else:
        o_ref[...] = hnew


def _t2(z, st, gamma, beta, pW=None, pb=None):
    final = pW is not None
    if not final:
        pW = jnp.zeros((H, D), _f32)
        pb = jnp.zeros((D,), _f32)
    return pl.pallas_call(
        functools.partial(_t2_body, final),
        grid=(NBLK,),
        in_specs=[
            pl.BlockSpec((BLK, H), lambda i: (i, 0)),
            pl.BlockSpec((2, H), lambda i: (0, 0)),
            pl.BlockSpec((1, H), lambda i: (0, 0)),
            pl.BlockSpec((1, H), lambda i: (0, 0)),
            pl.BlockSpec((H, D), lambda i: (0, 0)),
            pl.BlockSpec((1, D), lambda i: (0, 0)),
        ],
        out_specs=pl.BlockSpec((BLK, D), lambda i: (i, 0)),
        out_shape=jax.ShapeDtypeStruct((N, D), _f32),
    )(z, st, gamma.reshape(1, H), beta.reshape(1, H), pW, pb.reshape(1, D))


def kernel(x_user, x_item, Wself, Wneigh, b, bn_gamma, bn_beta, proj_W,
           proj_b, edge_ui, edge_iu):
    src_ui = edge_ui[0].astype(_i32)
    dst_ui = edge_ui[1].astype(_i32)
    src_iu = edge_iu[0].astype(_i32)
    dst_iu = edge_iu[1].astype(_i32)

    (bsrc_ui, bdst_ui, offs_ui, bsrc_iu, bdst_iu, offs_iu) = _bin_edges(
        src_ui, dst_ui, src_iu, dst_iu)

    zeros = jnp.zeros((ACC_ROWS, D), _f32)
    ones = jnp.ones((128, D), _f32)

    hu, hi = x_user, x_item
    dpart_ui = dpart_iu = None
    out_u = out_i = None
    for i in range(L):
        with_deg = i == 0
        part_ui, dp_ui = _aggregate(hu, bsrc_ui, bdst_ui, offs_ui, zeros,
                                    ones, with_deg)
        part_iu, dp_iu = _aggregate(hi, bsrc_iu, bdst_iu, offs_iu, zeros,
                                    ones, with_deg)
        if with_deg:
            dpart_ui, dpart_iu = dp_ui, dp_iu
        z_i, st_i = _t1(hi, part_ui, dpart_ui, Wself[i, 0], Wneigh[i, 0],
                        b[i, 0])
        z_u, st_u = _t1(hu, part_iu, dpart_iu, Wself[i, 1], Wneigh[i, 1],
                        b[i, 1])
        if i < L - 1:
            hu = _t2(z_u, st_u, bn_gamma[i, 0], bn_beta[i, 0])
            hi = _t2(z_i, st_i, bn_gamma[i, 1], bn_beta[i, 1])
        else:
            out_u = _t2(z_u, st_u, bn_gamma[i, 0], bn_beta[i, 0],
                        proj_W[0], proj_b[0])
            out_i = _t2(z_i, st_i, bn_gamma[i, 1], bn_beta[i, 1],
                        proj_W[1], proj_b[1])
    return (out_u, out_i)


# speedup vs baseline: 3.0759x; 1.0293x over previous
import dataclasses
import functools

import jax
import jax.numpy as jnp
from jax import lax
from jax.experimental import pallas as pl
from jax.experimental.pallas import tpu as pltpu
from jax.experimental.pallas import tpu_sc as plsc

N_USER = 50000
N_ITEM = 50000
N = 50000
E = 320000
D = 128
H = 128
L = 2
EPS = 1e-5

NC = 2
NS = 16
NW = NC * NS
EPW = E // NW
CSZ = 4096
SHIFT = 12
NCHUNK = 13
CAP = EPW + NCHUNK * 16
ACC_ROWS = 2 * CSZ + 128
DACC_ROWS = CSZ + 128
PN = 7 * 2 * CSZ
ZROWS = ACC_ROWS // NS
DZROWS = DACC_ROWS // NS
OROWS = CSZ // NS
OROWS2 = 2 * CSZ // NS

_i32 = jnp.int32
_f32 = jnp.float32


def _vmesh():
    return plsc.VectorSubcoreMesh(
        core_axis_name="c", subcore_axis_name="s", num_cores=NC, num_subcores=NS
    )


def _sc_params():
    cp = pltpu.CompilerParams()
    if "needs_layout_passes" in pltpu.CompilerParams.__dataclass_fields__:
        cp = dataclasses.replace(cp, needs_layout_passes=False)
    return cp


def _bin_kernel(src_ui, dst_ui, src_iu, dst_iu,
                bsrc_ui, bdst_ui, offs_ui, bsrc_iu, bdst_iu, offs_iu,
                src_v, dst_v, osrc_v, odst_v, offs_v, sem):
    wid = lax.axis_index("s") * NC + lax.axis_index("c")
    io16 = lax.iota(_i32, 16)
    for (srch, dsth, bsrch, bdsth, offsh) in (
        (src_ui, dst_ui, bsrc_ui, bdst_ui, offs_ui),
        (src_iu, dst_iu, bsrc_iu, bdst_iu, offs_iu),
    ):
        base = wid * EPW
        pltpu.async_copy(srch.at[pl.ds(base, EPW)], src_v, sem).wait()
        pltpu.async_copy(dsth.at[pl.ds(base, EPW)], dst_v, sem).wait()

        def cbody(i, accs):
            d = dst_v[pl.ds(i * 16, 16)]
            ch = lax.shift_right_logical(d, SHIFT)
            return tuple(
                accs[b] + jnp.where(ch == b, 1, 0).astype(_i32)
                for b in range(NCHUNK)
            )
        accs = lax.fori_loop(
            0, EPW // 16, cbody,
            tuple(jnp.zeros((16,), _i32) for _ in range(NCHUNK)),
        )
        cnts = [jnp.sum(a) for a in accs]

        offs = [jnp.zeros((), _i32)]
        for b in range(NCHUNK):
            pcnt = lax.bitwise_and(cnts[b] + 15, jnp.full((), ~15, _i32))
            offs.append(offs[b] + pcnt)

        def fbody(i, pos):
            s = src_v[pl.ds(i * 16, 16)]
            d = dst_v[pl.ds(i * 16, 16)]
            ch = lax.shift_right_logical(d, SHIFT)
            newpos = []
            for b in range(NCHUNK):
                m = ch == b
                mi = jnp.where(m, 1, 0).astype(_i32)
                posv = pos[b] + plsc.cumsum(mi) - 1
                posv = jnp.clip(posv, 0, CAP - 1)
                plsc.store_scatter(osrc_v, [posv], s, mask=m)
                plsc.store_scatter(odst_v, [posv], d, mask=m)
                newpos.append(pos[b] + jnp.sum(mi))
            return tuple(newpos)
        pos = lax.fori_loop(0, EPW // 16, fbody, tuple(offs[:NCHUNK]))

        for b in range(NCHUNK):
            posv = pos[b] + io16
            m = posv < offs[b + 1]
            posv = jnp.clip(posv, 0, CAP - 1)
            plsc.store_scatter(osrc_v, [posv], jnp.zeros((16,), _i32), mask=m)
            plsc.store_scatter(
                odst_v, [posv],
                jnp.full((16,), (b - b % 2 + 2) * CSZ, _i32), mask=m)

        ends = jnp.zeros((16,), _i32)
        for b in range(NCHUNK):
            ends = ends + jnp.where(io16 == b, offs[b + 1], 0).astype(_i32)
        offs_v[...] = ends

        pltpu.async_copy(osrc_v, bsrch.at[pl.ds(wid * CAP, CAP)], sem).wait()
        pltpu.async_copy(odst_v, bdsth.at[pl.ds(wid * CAP, CAP)], sem).wait()
        pltpu.async_copy(offs_v, offsh.at[pl.ds(wid * 16, 16)], sem).wait()


def _bin_edges(src_ui, dst_ui, src_iu, dst_iu):
    out = jax.ShapeDtypeStruct
    k = pl.kernel(
        _bin_kernel,
        out_type=[
            out((NW * CAP,), _i32), out((NW * CAP,), _i32),
            out((NW * 16,), _i32),
            out((NW * CAP,), _i32), out((NW * CAP,), _i32),
            out((NW * 16,), _i32),
        ],
        mesh=_vmesh(),
        scratch_types=[
            pltpu.VMEM((EPW,), _i32), pltpu.VMEM((EPW,), _i32),
            pltpu.VMEM((CAP,), _i32), pltpu.VMEM((CAP,), _i32),
            pltpu.VMEM((16,), _i32), pltpu.SemaphoreType.DMA,
        ],
        compiler_params=_sc_params(),
    )
    return k(src_ui, dst_ui, src_iu, dst_iu)


def _agg_kernel(with_deg, h_hbm, bsrc_hbm, bdst_hbm, offs_hbm, zeros_hbm,
                ones_hbm, *refs):
    if with_deg:
        part_hbm, dpart_hbm = refs[0], refs[1]
        scr = refs[2:]
    else:
        part_hbm = refs[0]
        dpart_hbm = None
        scr = refs[1:]
    if with_deg:
        (acc, dacc, src_a, src_b, dst_a, dst_b, ldst_a, ldst_b,
         rows_a, rows_b, ones_v, ldst_t, rows_t, ones_t, offs_v,
         sem, sem2, sem3, sem4) = scr
    else:
        (acc, src_a, src_b, dst_a, dst_b, ldst_a, ldst_b,
         rows_a, rows_b, ones_v, ldst_t, rows_t, ones_t, offs_v,
         sem, sem2, sem3, sem4) = scr
        dacc = None

    core = lax.axis_index("c")
    tl = lax.axis_index("s")
    wid = tl * NC + core
    io16 = lax.iota(_i32, 16)

    if with_deg:
        pltpu.sync_copy(ones_hbm, ones_v)
        pltpu.sync_copy(ones_hbm.at[pl.ds(0, 16)], ones_t)

    pltpu.async_copy(offs_hbm.at[pl.ds(wid * 16, 16)], offs_v, sem).wait()
    ovec = offs_v[...]
    ends = [jnp.sum(jnp.where(io16 == b, ovec, 0).astype(_i32))
            for b in range(NCHUNK)]
    starts = [jnp.zeros((), _i32)] + ends[: NCHUNK - 1]

    if with_deg:
        chunks = [(b, starts[b], ends[b], b * CSZ, OROWS) for b in range(NCHUNK)]
    else:
        chunks = [(c, starts[2 * c], ends[min(2 * c + 1, NCHUNK - 1)],
                   c * 2 * CSZ, OROWS2) for c in range((NCHUNK + 1) // 2)]

    for (_, s0, e1, cb, orows) in chunks:
        plsc.subcore_barrier()
        zr = DZROWS if with_deg else ZROWS
        pltpu.async_copy(zeros_hbm.at[pl.ds(tl * zr, zr)],
                         acc.at[pl.ds(tl * zr, zr)], sem).wait()
        if with_deg:
            pltpu.async_copy(zeros_hbm.at[pl.ds(tl * DZROWS, DZROWS)],
                             dacc.at[pl.ds(tl * DZROWS, DZROWS)], sem).wait()
        plsc.subcore_barrier()

        n = e1 - s0
        npair = lax.bitwise_and(n, jnp.full((), ~255, _i32))
        nmain = lax.bitwise_and(n, jnp.full((), ~127, _i32))
        cbase = jnp.full((), cb, _i32)
        dclamp = jnp.full((16,), CSZ, _i32)

        def pbody(g, _):
            offa = pl.multiple_of(wid * CAP + s0 + g * 256, 16)
            offb = pl.multiple_of(offa + 128, 16)
            ia = pltpu.async_copy(bsrc_hbm.at[pl.ds(offa, 128)], src_a, sem)
            ib = pltpu.async_copy(bsrc_hbm.at[pl.ds(offb, 128)], src_b, sem2)
            da = pltpu.async_copy(bdst_hbm.at[pl.ds(offa, 128)], dst_a, sem3)
            db = pltpu.async_copy(bdst_hbm.at[pl.ds(offb, 128)], dst_b, sem4)
            ia.wait()
            ga = pltpu.async_copy(h_hbm.at[src_a], rows_a, sem)
            ib.wait()
            gb = pltpu.async_copy(h_hbm.at[src_b], rows_b, sem2)
            da.wait()
            db.wait()

            @pl.loop(0, 128, step=16)
            def _(k):
                la = dst_a[pl.ds(k, 16)] - cbase
                lb = dst_b[pl.ds(k, 16)] - cbase
                if with_deg:
                    la = jnp.minimum(la, dclamp)
                    lb = jnp.minimum(lb, dclamp)
                ldst_a[pl.ds(k, 16)] = la
                ldst_b[pl.ds(k, 16)] = lb

            ga.wait()
            pltpu.sync_copy(rows_a, acc.at[ldst_a], add=True)
            if with_deg:
                pltpu.sync_copy(ones_v, dacc.at[ldst_a], add=True)
            gb.wait()
            pltpu.sync_copy(rows_b, acc.at[ldst_b], add=True)
            if with_deg:
                pltpu.sync_copy(ones_v, dacc.at[ldst_b], add=True)
            return 0
        lax.fori_loop(0, npair // 256, pbody, 0)

        @pl.when(nmain > npair)
        def _():
            offa = pl.multiple_of(wid * CAP + s0 + npair, 16)
            pltpu.async_copy(bsrc_hbm.at[pl.ds(offa, 128)], src_a, sem).wait()
            pltpu.async_copy(bdst_hbm.at[pl.ds(offa, 128)], dst_a, sem).wait()

            @pl.loop(0, 128, step=16)
            def _(k):
                la = dst_a[pl.ds(k, 16)] - cbase
                if with_deg:
                    la = jnp.minimum(la, dclamp)
                ldst_a[pl.ds(k, 16)] = la

            pltpu.async_copy(h_hbm.at[src_a], rows_a, sem).wait()
            pltpu.sync_copy(rows_a, acc.at[ldst_a], add=True)
            if with_deg:
                pltpu.sync_copy(ones_v, dacc.at[ldst_a], add=True)

        def tbody(g, _):
            off = pl.multiple_of(wid * CAP + s0 + nmain + g * 16, 16)
            pltpu.async_copy(bsrc_hbm.at[pl.ds(off, 16)], src_a.at[pl.ds(0, 16)],
                             sem).wait()
            pltpu.async_copy(bdst_hbm.at[pl.ds(off, 16)], dst_a.at[pl.ds(0, 16)],
                             sem).wait()
            lt = dst_a[pl.ds(0, 16)] - cbase
            if with_deg:
                lt = jnp.minimum(lt, dclamp)
            ldst_t[...] = lt
            pltpu.async_copy(h_hbm.at[src_a.at[pl.ds(0, 16)]], rows_t,
                             sem).wait()
            pltpu.sync_copy(rows_t, acc.at[ldst_t], add=True)
            if with_deg:
                pltpu.sync_copy(ones_t, dacc.at[ldst_t], add=True)
            return 0
        lax.fori_loop(0, (n - nmain) // 16, tbody, 0)

        plsc.subcore_barrier()
        obase = cb + tl * orows
        pltpu.async_copy(acc.at[pl.ds(tl * orows, orows)],
                         part_hbm.at[core, pl.ds(obase, orows)], sem).wait()
        if with_deg:
            pltpu.async_copy(dacc.at[pl.ds(tl * OROWS, OROWS)],
                             dpart_hbm.at[core, pl.ds(cb + tl * OROWS, OROWS)],
                             sem).wait()


def _aggregate(h, bsrc, bdst, offs, zeros, ones, with_deg):
    out = jax.ShapeDtypeStruct
    outs = [out((NC, PN, D), _f32)]
    if with_deg:
        outs.append(out((NC, PN, D), _f32))
    k = pl.kernel(
        functools.partial(_agg_kernel, with_deg),
        out_type=outs,
        mesh=_vmesh(),
        scratch_types=(
            ([pltpu.VMEM_SHARED((DACC_ROWS, D), _f32),
              pltpu.VMEM_SHARED((DACC_ROWS, D), _f32)] if with_deg
             else [pltpu.VMEM_SHARED((ACC_ROWS, D), _f32)])
            + [
            pltpu.VMEM((128,), _i32), pltpu.VMEM((128,), _i32),
            pltpu.VMEM((128,), _i32), pltpu.VMEM((128,), _i32),
            pltpu.VMEM((128,), _i32), pltpu.VMEM((128,), _i32),
            pltpu.VMEM((128, D), _f32), pltpu.VMEM((128, D), _f32),
            pltpu.VMEM((128, D), _f32),
            pltpu.VMEM((16,), _i32), pltpu.VMEM((16, D), _f32),
            pltpu.VMEM((16, D), _f32),
            pltpu.VMEM((16,), _i32),
            pltpu.SemaphoreType.DMA, pltpu.SemaphoreType.DMA,
            pltpu.SemaphoreType.DMA, pltpu.SemaphoreType.DMA,
        ]),
        compiler_params=_sc_params(),
    )
    res = k(h, bsrc, bdst, offs, zeros, ones)
    return res if with_deg else (res[0], None)


BLK = 1000
NBLK = N // BLK


def _t1_body(h_ref, p_ref, d_ref, ws_ref, wn_ref, b_ref, z_ref, st_ref):
    a = p_ref[0] + p_ref[1]
    deg = d_ref[0][:, 0:1] + d_ref[1][:, 0:1]
    hn = a * (1.0 / jnp.maximum(deg, 1.0))
    z = (jnp.dot(h_ref[...], ws_ref[...], preferred_element_type=_f32)
         + jnp.dot(hn, wn_ref[...], preferred_element_type=_f32)
         + b_ref[...])
    z_ref[...] = z
    zs = jnp.sum(z, axis=0, keepdims=True)
    zss = jnp.sum(z * z, axis=0, keepdims=True)
    i = pl.program_id(0)

    @pl.when(i == 0)
    def _():
        st_ref[0:1, :] = zs
        st_ref[1:2, :] = zss

    @pl.when(i > 0)
    def _():
        st_ref[0:1, :] = st_ref[0:1, :] + zs
        st_ref[1:2, :] = st_ref[1:2, :] + zss


def _t1(h, part, dpart, Ws, Wn, bias):
    return pl.pallas_call(
        _t1_body,
        grid=(NBLK,),
        in_specs=[
            pl.BlockSpec((BLK, D), lambda i: (i, 0)),
            pl.BlockSpec((NC, BLK, D), lambda i: (0, i, 0)),
            pl.BlockSpec((NC, BLK, D), lambda i: (0, i, 0)),
            pl.BlockSpec((D, H), lambda i: (0, 0)),
            pl.BlockSpec((D, H), lambda i: (0, 0)),
            pl.BlockSpec((1, H), lambda i: (0, 0)),
        ],
        out_specs=[
            pl.BlockSpec((BLK, H), lambda i: (i, 0)),
            pl.BlockSpec((2, H), lambda i: (0, 0)),
        ],
        out_shape=[
            jax.ShapeDtypeStruct((N, H), _f32),
            jax.ShapeDtypeStruct((2, H), _f32),
        ],
    )(h, part, dpart, Ws, Wn, bias.reshape(1, H))


def _t2_body(final, z_ref, st_ref, g_ref, be_ref, pw_ref, pb_ref, o_ref):
    mu = st_ref[0:1, :] * (1.0 / N)
    var = st_ref[1:2, :] * (1.0 / N) - mu * mu
    scale = g_ref[...] * lax.rsqrt(var + EPS)
    hnew = jnp.maximum((z_ref[...] - mu) * scale + be_ref[...], 0.0)
    if final:
        o_ref[...] = (jnp.dot(hnew, pw_ref[...], preferred_element_type=_f32)
                      + pb_ref[...])
    else:
        o_ref[...] = hnew


def _t2(z, st, gamma, beta, pW=None, pb=None):
    final = pW is not None
    if not final:
        pW = jnp.zeros((H, D), _f32)
        pb = jnp.zeros((D,), _f32)
    return pl.pallas_call(
        functools.partial(_t2_body, final),
        grid=(NBLK,),
        in_specs=[
            pl.BlockSpec((BLK, H), lambda i: (i, 0)),
            pl.BlockSpec((2, H), lambda i: (0, 0)),
            pl.BlockSpec((1, H), lambda i: (0, 0)),
            pl.BlockSpec((1, H), lambda i: (0, 0)),
            pl.BlockSpec((H, D), lambda i: (0, 0)),
            pl.BlockSpec((1, D), lambda i: (0, 0)),
        ],
        out_specs=pl.BlockSpec((BLK, D), lambda i: (i, 0)),
        out_shape=jax.ShapeDtypeStruct((N, D), _f32),
    )(z, st, gamma.reshape(1, H), beta.reshape(1, H), pW, pb.reshape(1, D))


def kernel(x_user, x_item, Wself, Wneigh, b, bn_gamma, bn_beta, proj_W,
           proj_b, edge_ui, edge_iu):
    src_ui = edge_ui[0].astype(_i32)
    dst_ui = edge_ui[1].astype(_i32)
    src_iu = edge_iu[0].astype(_i32)
    dst_iu = edge_iu[1].astype(_i32)

    (bsrc_ui, bdst_ui, offs_ui, bsrc_iu, bdst_iu, offs_iu) = _bin_edges(
        src_ui, dst_ui, src_iu, dst_iu)

    zeros = jnp.zeros((ACC_ROWS, D), _f32)
    ones = jnp.ones((128, D), _f32)

    hu, hi = x_user, x_item
    dpart_ui = dpart_iu = None
    out_u = out_i = None
    for i in range(L):
        with_deg = i == 0
        part_ui, dp_ui = _aggregate(hu, bsrc_ui, bdst_ui, offs_ui, zeros,
                                    ones, with_deg)
        part_iu, dp_iu = _aggregate(hi, bsrc_iu, bdst_iu, offs_iu, zeros,
                                    ones, with_deg)
        if with_deg:
            dpart_ui, dpart_iu = dp_ui, dp_iu
        z_i, st_i = _t1(hi, part_ui, dpart_ui, Wself[i, 0], Wneigh[i, 0],
                        b[i, 0])
        z_u, st_u = _t1(hu, part_iu, dpart_iu, Wself[i, 1], Wneigh[i, 1],
                        b[i, 1])
        if i < L - 1:
            hu = _t2(z_u, st_u, bn_gamma[i, 0], bn_beta[i, 0])
            hi = _t2(z_i, st_i, bn_gamma[i, 1], bn_beta[i, 1])
        else:
            out_u = _t2(z_u, st_u, bn_gamma[i, 0], bn_beta[i, 0],
                        proj_W[0], proj_b[0])
            out_i = _t2(z_i, st_i, bn_gamma[i, 1], bn_beta[i, 1],
                        proj_W[1], proj_b[1])
    return (out_u, out_i)
